# Initial kernel scaffold; baseline (speedup 1.0000x reference)
import jax
import jax.numpy as jnp
from jax.experimental import pallas as pl


def kernel(embed_id, weight):
    raise NotImplementedError("write your pallas kernel here")


# speedup vs baseline: 2.2471x; 2.2471x over previous
import functools

import jax
import jax.numpy as jnp
from jax import lax
from jax.experimental import pallas as pl
from jax.experimental.pallas import tpu as pltpu
from jax.experimental.pallas import tpu_sc as plsc

NUM_TOKENS = 8192
D = 256
B = 16384

NC = 2
NS = 16
NW = NC * NS
B_PER_W = B // NW
CH = 128
N_CHUNKS = B_PER_W // CH


def _transpose_body(w_ref, t_ref):
    t_ref[...] = w_ref[...].T


def _transpose_weight(weight):
    blk = 512
    return pl.pallas_call(
        _transpose_body,
        grid=(NUM_TOKENS // blk,),
        in_specs=[pl.BlockSpec((D, blk), lambda i: (0, i))],
        out_specs=pl.BlockSpec((blk, D), lambda i: (i, 0)),
        out_shape=jax.ShapeDtypeStruct((NUM_TOKENS, D), jnp.float32),
    )(weight)


def _sc_gather_body(table_hbm, idx_hbm, out_hbm, idx_v, bufs, sem0, sem1):
    sems = (sem0, sem1)
    wid = lax.axis_index("s") * NC + lax.axis_index("c")
    base = wid * B_PER_W
    pltpu.sync_copy(idx_hbm.at[wid], idx_v)
    copies = [None, None]
    for c in range(N_CHUNKS):
        b = c % 2
        copies[b] = pltpu.async_copy(
            table_hbm.at[idx_v.at[c]], bufs.at[b], sems[b])
        if c > 0:
            pb = (c - 1) % 2
            copies[pb].wait()
            pltpu.sync_copy(
                bufs.at[pb],
                out_hbm.at[pl.ds(base + (c - 1) * CH, CH)])
    lb = (N_CHUNKS - 1) % 2
    copies[lb].wait()
    pltpu.sync_copy(
        bufs.at[lb], out_hbm.at[pl.ds(base + (N_CHUNKS - 1) * CH, CH)])


@functools.partial(
    pl.kernel,
    mesh=plsc.VectorSubcoreMesh(core_axis_name="c", subcore_axis_name="s"),
    out_type=jax.ShapeDtypeStruct((B, D), jnp.float32),
    scratch_types=[
        pltpu.VMEM((N_CHUNKS, CH), jnp.int32),
        pltpu.VMEM((2, CH, D), jnp.float32),
        pltpu.SemaphoreType.DMA,
        pltpu.SemaphoreType.DMA,
    ],
)
def _sc_gather(table_hbm, idx_hbm, out_hbm, idx_v, bufs, sem0, sem1):
    _sc_gather_body(table_hbm, idx_hbm, out_hbm, idx_v, bufs, sem0, sem1)


def kernel(embed_id, weight):
    shape = embed_id.shape
    idx = embed_id.reshape(NW, N_CHUNKS, CH).astype(jnp.int32)
    table = _transpose_weight(weight)
    out = _sc_gather(table, idx)
    return out.reshape(*shape, D)


# speedup vs baseline: 2.2472x; 1.0000x over previous
import functools

import jax
import jax.numpy as jnp
from jax import lax
from jax.experimental import pallas as pl
from jax.experimental.pallas import tpu as pltpu
from jax.experimental.pallas import tpu_sc as plsc

NUM_TOKENS = 8192
D = 256
B = 16384

NC = 2
NS = 16
NW = NC * NS
B_PER_W = B // NW
CH = 128
N_CHUNKS = B_PER_W // CH


def _transpose_body(w_ref, t_ref):
    t_ref[...] = w_ref[...].T


def _transpose_weight(weight):
    blk = 512
    return pl.pallas_call(
        _transpose_body,
        grid=(NUM_TOKENS // blk,),
        in_specs=[pl.BlockSpec((D, blk), lambda i: (0, i))],
        out_specs=pl.BlockSpec((blk, D), lambda i: (i, 0)),
        out_shape=jax.ShapeDtypeStruct((NUM_TOKENS, D), jnp.float32),
    )(weight)


NBUF = 3


def _sc_gather_body(table_hbm, idx_hbm, out_hbm, idx_v, bufs, gsems, wsems):
    wid = lax.axis_index("s") * NC + lax.axis_index("c")
    base = wid * B_PER_W
    pltpu.sync_copy(idx_hbm.at[wid], idx_v)
    def gather(c):
        return pltpu.async_copy(
            table_hbm.at[idx_v.at[c]], bufs.at[c % NBUF], gsems[c % NBUF])

    def write(c):
        return pltpu.async_copy(
            bufs.at[c % NBUF], out_hbm.at[pl.ds(base + c * CH, CH)],
            wsems[c % NBUF])

    gathers = [None] * N_CHUNKS
    writes = [None] * N_CHUNKS
    for c in range(min(NBUF, N_CHUNKS)):
        gathers[c] = gather(c)
    for c in range(min(NBUF, N_CHUNKS)):
        gathers[c].wait()
        writes[c] = write(c)
    for c in range(NBUF, N_CHUNKS):
        writes[c - NBUF].wait()
        gathers[c] = gather(c)
        gathers[c].wait()
        writes[c] = write(c)
    for c in range(max(0, N_CHUNKS - NBUF), N_CHUNKS):
        writes[c].wait()


@functools.partial(
    pl.kernel,
    mesh=plsc.VectorSubcoreMesh(core_axis_name="c", subcore_axis_name="s"),
    out_type=jax.ShapeDtypeStruct((B, D), jnp.float32),
    scratch_types=[
        pltpu.VMEM((N_CHUNKS, CH), jnp.int32),
        pltpu.VMEM((NBUF, CH, D), jnp.float32),
        pltpu.SemaphoreType.DMA,
        pltpu.SemaphoreType.DMA,
        pltpu.SemaphoreType.DMA,
        pltpu.SemaphoreType.DMA,
        pltpu.SemaphoreType.DMA,
        pltpu.SemaphoreType.DMA,
    ],
)
def _sc_gather(table_hbm, idx_hbm, out_hbm, idx_v, bufs,
               g0, g1, g2, w0, w1, w2):
    _sc_gather_body(table_hbm, idx_hbm, out_hbm, idx_v, bufs,
                    (g0, g1, g2), (w0, w1, w2))


def kernel(embed_id, weight):
    shape = embed_id.shape
    idx = embed_id.reshape(NW, N_CHUNKS, CH).astype(jnp.int32)
    table = _transpose_weight(weight)
    out = _sc_gather(table, idx)
    return out.reshape(*shape, D)


# speedup vs baseline: 2.2601x; 1.0058x over previous
import functools

import jax
import jax.numpy as jnp
from jax import lax
from jax.experimental import pallas as pl
from jax.experimental.pallas import tpu as pltpu
from jax.experimental.pallas import tpu_sc as plsc

NUM_TOKENS = 8192
D = 256
B = 16384

NC = 2
NS = 16
NW = NC * NS
B_PER_W = B // NW
CH = 64
N_CHUNKS = B_PER_W // CH


def _transpose_body(w_ref, t_ref):
    t_ref[...] = w_ref[...].T


def _transpose_weight(weight):
    blk = 512
    return pl.pallas_call(
        _transpose_body,
        grid=(NUM_TOKENS // blk,),
        in_specs=[pl.BlockSpec((D, blk), lambda i: (0, i))],
        out_specs=pl.BlockSpec((blk, D), lambda i: (i, 0)),
        out_shape=jax.ShapeDtypeStruct((NUM_TOKENS, D), jnp.float32),
    )(weight)


NBUF = 6


def _sc_gather_body(table_hbm, idx_hbm, out_hbm, idx_v, bufs, gsems, wsems):
    wid = lax.axis_index("s") * NC + lax.axis_index("c")
    base = wid * B_PER_W
    pltpu.sync_copy(idx_hbm.at[wid], idx_v)
    def gather(c):
        return pltpu.async_copy(
            table_hbm.at[idx_v.at[c]], bufs.at[c % NBUF], gsems[c % NBUF])

    def write(c):
        return pltpu.async_copy(
            bufs.at[c % NBUF], out_hbm.at[pl.ds(base + c * CH, CH)],
            wsems[c % NBUF])

    gathers = [None] * N_CHUNKS
    writes = [None] * N_CHUNKS
    for c in range(min(NBUF, N_CHUNKS)):
        gathers[c] = gather(c)
    for c in range(min(NBUF, N_CHUNKS)):
        gathers[c].wait()
        writes[c] = write(c)
    for c in range(NBUF, N_CHUNKS):
        writes[c - NBUF].wait()
        gathers[c] = gather(c)
        gathers[c].wait()
        writes[c] = write(c)
    for c in range(max(0, N_CHUNKS - NBUF), N_CHUNKS):
        writes[c].wait()


@functools.partial(
    pl.kernel,
    mesh=plsc.VectorSubcoreMesh(core_axis_name="c", subcore_axis_name="s"),
    out_type=jax.ShapeDtypeStruct((B, D), jnp.float32),
    scratch_types=(
        [pltpu.VMEM((N_CHUNKS, CH), jnp.int32),
         pltpu.VMEM((NBUF, CH, D), jnp.float32)]
        + [pltpu.SemaphoreType.DMA] * (2 * NBUF)
    ),
)
def _sc_gather(table_hbm, idx_hbm, out_hbm, idx_v, bufs, *sems):
    _sc_gather_body(table_hbm, idx_hbm, out_hbm, idx_v, bufs,
                    sems[:NBUF], sems[NBUF:])


def kernel(embed_id, weight):
    shape = embed_id.shape
    idx = embed_id.reshape(NW, N_CHUNKS, CH).astype(jnp.int32)
    table = _transpose_weight(weight)
    out = _sc_gather(table, idx)
    return out.reshape(*shape, D)


# speedup vs baseline: 2.4702x; 1.0930x over previous
import functools

import jax
import jax.numpy as jnp
from jax import lax
from jax.experimental import pallas as pl
from jax.experimental.pallas import tpu as pltpu
from jax.experimental.pallas import tpu_sc as plsc

NUM_TOKENS = 8192
D = 256
B = 16384

NC = 2
NS = 16
NW = NC * NS
B_PER_W = B // NW
CH = 64
N_CHUNKS = B_PER_W // CH


_TBLK = 1024


def _transpose_body(w_ref, t_ref):
    t_ref[...] = w_ref[...].T


def _transpose_weight(weight):
    return pl.pallas_call(
        _transpose_body,
        grid=(NUM_TOKENS // _TBLK,),
        in_specs=[pl.BlockSpec((D, _TBLK), lambda i: (0, i))],
        out_specs=pl.BlockSpec((_TBLK, D), lambda i: (i, 0)),
        out_shape=jax.ShapeDtypeStruct((NUM_TOKENS, D), jnp.float32),
    )(weight)


NBUF = 6


def _sc_gather_body(table_hbm, idx_hbm, out_hbm, idx_v, bufs, gsems, wsems):
    wid = lax.axis_index("s") * NC + lax.axis_index("c")
    base = wid * B_PER_W
    pltpu.sync_copy(idx_hbm.at[wid], idx_v)
    def gather(c):
        return pltpu.async_copy(
            table_hbm.at[idx_v.at[c]], bufs.at[c % NBUF], gsems[c % NBUF])

    def write(c):
        return pltpu.async_copy(
            bufs.at[c % NBUF], out_hbm.at[pl.ds(base + c * CH, CH)],
            wsems[c % NBUF])

    gathers = [None] * N_CHUNKS
    writes = [None] * N_CHUNKS
    for c in range(min(NBUF, N_CHUNKS)):
        gathers[c] = gather(c)
    for c in range(min(NBUF, N_CHUNKS)):
        gathers[c].wait()
        writes[c] = write(c)
    for c in range(NBUF, N_CHUNKS):
        writes[c - NBUF].wait()
        gathers[c] = gather(c)
        gathers[c].wait()
        writes[c] = write(c)
    for c in range(max(0, N_CHUNKS - NBUF), N_CHUNKS):
        writes[c].wait()


@functools.partial(
    pl.kernel,
    mesh=plsc.VectorSubcoreMesh(core_axis_name="c", subcore_axis_name="s"),
    out_type=jax.ShapeDtypeStruct((B, D), jnp.float32),
    scratch_types=(
        [pltpu.VMEM((N_CHUNKS, CH), jnp.int32),
         pltpu.VMEM((NBUF, CH, D), jnp.float32)]
        + [pltpu.SemaphoreType.DMA] * (2 * NBUF)
    ),
)
def _sc_gather(table_hbm, idx_hbm, out_hbm, idx_v, bufs, *sems):
    _sc_gather_body(table_hbm, idx_hbm, out_hbm, idx_v, bufs,
                    sems[:NBUF], sems[NBUF:])


def kernel(embed_id, weight):
    shape = embed_id.shape
    idx = embed_id.reshape(NW, N_CHUNKS, CH).astype(jnp.int32)
    table = _transpose_weight(weight)
    out = _sc_gather(table, idx)
    return out.reshape(*shape, D)


# speedup vs baseline: 2.6047x; 1.0544x over previous
import functools

import jax
import jax.numpy as jnp
from jax import lax
from jax.experimental import pallas as pl
from jax.experimental.pallas import tpu as pltpu
from jax.experimental.pallas import tpu_sc as plsc

NUM_TOKENS = 8192
D = 256
B = 16384

NC = 2
NS = 16
NW = NC * NS
B_PER_W = B // NW
CH = 64
N_CHUNKS = B_PER_W // CH


_TBLK = 2048


def _transpose_body(w_ref, t_ref):
    t_ref[...] = w_ref[...].T


def _transpose_weight(weight):
    return pl.pallas_call(
        _transpose_body,
        grid=(NUM_TOKENS // _TBLK,),
        in_specs=[pl.BlockSpec((D, _TBLK), lambda i: (0, i))],
        out_specs=pl.BlockSpec((_TBLK, D), lambda i: (i, 0)),
        out_shape=jax.ShapeDtypeStruct((NUM_TOKENS, D), jnp.float32),
    )(weight)


NBUF = 6


def _sc_gather_body(table_hbm, idx_hbm, out_hbm, idx_v, bufs, gsems, wsems):
    wid = lax.axis_index("s") * NC + lax.axis_index("c")
    base = wid * B_PER_W
    pltpu.sync_copy(idx_hbm.at[wid], idx_v)
    def gather(c):
        return pltpu.async_copy(
            table_hbm.at[idx_v.at[c]], bufs.at[c % NBUF], gsems[c % NBUF])

    def write(c):
        return pltpu.async_copy(
            bufs.at[c % NBUF], out_hbm.at[pl.ds(base + c * CH, CH)],
            wsems[c % NBUF])

    gathers = [None] * N_CHUNKS
    writes = [None] * N_CHUNKS
    for c in range(min(NBUF, N_CHUNKS)):
        gathers[c] = gather(c)
    for c in range(min(NBUF, N_CHUNKS)):
        gathers[c].wait()
        writes[c] = write(c)
    for c in range(NBUF, N_CHUNKS):
        writes[c - NBUF].wait()
        gathers[c] = gather(c)
        gathers[c].wait()
        writes[c] = write(c)
    for c in range(max(0, N_CHUNKS - NBUF), N_CHUNKS):
        writes[c].wait()


@functools.partial(
    pl.kernel,
    mesh=plsc.VectorSubcoreMesh(core_axis_name="c", subcore_axis_name="s"),
    out_type=jax.ShapeDtypeStruct((B, D), jnp.float32),
    scratch_types=(
        [pltpu.VMEM((N_CHUNKS, CH), jnp.int32),
         pltpu.VMEM((NBUF, CH, D), jnp.float32)]
        + [pltpu.SemaphoreType.DMA] * (2 * NBUF)
    ),
)
def _sc_gather(table_hbm, idx_hbm, out_hbm, idx_v, bufs, *sems):
    _sc_gather_body(table_hbm, idx_hbm, out_hbm, idx_v, bufs,
                    sems[:NBUF], sems[NBUF:])


def kernel(embed_id, weight):
    shape = embed_id.shape
    idx = embed_id.reshape(NW, N_CHUNKS, CH).astype(jnp.int32)
    table = _transpose_weight(weight)
    out = _sc_gather(table, idx)
    return out.reshape(*shape, D)


# speedup vs baseline: 2.6991x; 1.0362x over previous
import functools

import jax
import jax.numpy as jnp
from jax import lax
from jax.experimental import pallas as pl
from jax.experimental.pallas import tpu as pltpu
from jax.experimental.pallas import tpu_sc as plsc

NUM_TOKENS = 8192
D = 256
B = 16384

NC = 2
NS = 16
NW = NC * NS
B_PER_W = B // NW
CH = 64
N_CHUNKS = B_PER_W // CH


_TBLK = 4096


def _transpose_body(w_ref, t_ref):
    t_ref[...] = w_ref[...].T


def _transpose_weight(weight):
    return pl.pallas_call(
        _transpose_body,
        grid=(NUM_TOKENS // _TBLK,),
        in_specs=[pl.BlockSpec((D, _TBLK), lambda i: (0, i))],
        out_specs=pl.BlockSpec((_TBLK, D), lambda i: (i, 0)),
        out_shape=jax.ShapeDtypeStruct((NUM_TOKENS, D), jnp.float32),
    )(weight)


NBUF = 6


def _sc_gather_body(table_hbm, idx_hbm, out_hbm, idx_v, bufs, gsems, wsems):
    wid = lax.axis_index("s") * NC + lax.axis_index("c")
    base = wid * B_PER_W
    pltpu.sync_copy(idx_hbm.at[wid], idx_v)
    def gather(c):
        return pltpu.async_copy(
            table_hbm.at[idx_v.at[c]], bufs.at[c % NBUF], gsems[c % NBUF])

    def write(c):
        return pltpu.async_copy(
            bufs.at[c % NBUF], out_hbm.at[pl.ds(base + c * CH, CH)],
            wsems[c % NBUF])

    gathers = [None] * N_CHUNKS
    writes = [None] * N_CHUNKS
    for c in range(min(NBUF, N_CHUNKS)):
        gathers[c] = gather(c)
    for c in range(min(NBUF, N_CHUNKS)):
        gathers[c].wait()
        writes[c] = write(c)
    for c in range(NBUF, N_CHUNKS):
        writes[c - NBUF].wait()
        gathers[c] = gather(c)
        gathers[c].wait()
        writes[c] = write(c)
    for c in range(max(0, N_CHUNKS - NBUF), N_CHUNKS):
        writes[c].wait()


@functools.partial(
    pl.kernel,
    mesh=plsc.VectorSubcoreMesh(core_axis_name="c", subcore_axis_name="s"),
    out_type=jax.ShapeDtypeStruct((B, D), jnp.float32),
    scratch_types=(
        [pltpu.VMEM((N_CHUNKS, CH), jnp.int32),
         pltpu.VMEM((NBUF, CH, D), jnp.float32)]
        + [pltpu.SemaphoreType.DMA] * (2 * NBUF)
    ),
)
def _sc_gather(table_hbm, idx_hbm, out_hbm, idx_v, bufs, *sems):
    _sc_gather_body(table_hbm, idx_hbm, out_hbm, idx_v, bufs,
                    sems[:NBUF], sems[NBUF:])


def kernel(embed_id, weight):
    shape = embed_id.shape
    idx = embed_id.reshape(NW, N_CHUNKS, CH).astype(jnp.int32)
    table = _transpose_weight(weight)
    out = _sc_gather(table, idx)
    return out.reshape(*shape, D)
